# baseline (speedup 1.0000x reference)
import jax
import jax.numpy as jnp
from jax.experimental import pallas as pl


def _add_body(m_ref, md_ref, r_ref, rd_ref, mo_ref, ro_ref):
    mo_ref[...] = m_ref[...] + md_ref[...]
    ro_ref[...] = r_ref[...] + rd_ref[...]


def kernel(means, scales, rot, means_def, rot_def):
    n = means.shape[0]
    nm = n * 3
    nr = n * 4
    grid = 12
    bm = pl.cdiv(pl.cdiv(nm, grid), 1024) * 1024
    br = pl.cdiv(pl.cdiv(nr, grid), 1024) * 1024
    bs3 = pl.BlockSpec((bm,), lambda i: (i,))
    bs4 = pl.BlockSpec((br,), lambda i: (i,))
    mo, ro = pl.pallas_call(
        _add_body,
        grid=(grid,),
        in_specs=[bs3, bs3, bs4, bs4],
        out_specs=[bs3, bs4],
        out_shape=[
            jax.ShapeDtypeStruct((nm,), means.dtype),
            jax.ShapeDtypeStruct((nr,), rot.dtype),
        ],
    )(means.reshape(nm), means_def.reshape(nm), rot.reshape(nr), rot_def.reshape(nr))
    return (mo.reshape(n, 3), scales, ro.reshape(n, 4))


# speedup vs baseline: 188.0495x; 188.0495x over previous
import jax
import jax.numpy as jnp
from jax.experimental import pallas as pl


def _add_body(m_ref, md_ref, r_ref, rd_ref, mo_ref, ro_ref):
    mo_ref[...] = m_ref[...] + md_ref[...]
    ro_ref[...] = r_ref[...] + rd_ref[...]


def kernel(means, scales, rot, means_def, rot_def):
    n = means.shape[0]
    B = 32768
    g = pl.cdiv(n, B)
    bs3 = pl.BlockSpec((3, B), lambda i: (0, i))
    bs4 = pl.BlockSpec((4, B), lambda i: (0, i))
    mo_t, ro_t = pl.pallas_call(
        _add_body,
        grid=(g,),
        in_specs=[bs3, bs3, bs4, bs4],
        out_specs=[bs3, bs4],
        out_shape=[
            jax.ShapeDtypeStruct((3, n), means.dtype),
            jax.ShapeDtypeStruct((4, n), rot.dtype),
        ],
    )(means.T, means_def.T, rot.T, rot_def.T)
    return (mo_t.T, scales, ro_t.T)
